# baseline (speedup 1.0000x reference)
import math

import jax
import jax.numpy as jnp
import numpy as np
from jax.experimental import pallas as pl

_S = 1024
_K = 32
_OUT_DIM = 256
_IN_DIM = 3
_SIGMA = 0.26
_BASELINE = 0.1
_SCALING = 10.0
_EPS = 1e-06

_fd = math.ceil(_OUT_DIM / _IN_DIM)
_FEAT_NUM = _fd * _IN_DIM
_OUT_IDX = np.linspace(0, _FEAT_NUM - 1, _OUT_DIM).astype(np.int32)
_FEAT_VAL = np.linspace(-1.0, 1.0, _fd + 2)[1:-1].reshape(1, -1).astype(np.float32)


def _idx_pts(points, idx):
    return jax.vmap(lambda p, i: p[i])(points, idx)


def _sqdist(src, dst):
    dist = -2.0 * jnp.matmul(src, jnp.transpose(dst, (0, 2, 1)))
    dist = dist + jnp.sum(src ** 2, -1)[:, :, None]
    dist = dist + jnp.sum(dst ** 2, -1)[:, None, :]
    return dist


def _fps_j(xyz, s):
    B, N, _ = xyz.shape
    def step(carry, _):
        dist, farthest = carry
        centroid = jnp.take_along_axis(xyz, farthest[:, None, None], axis=1)
        d = jnp.sum((xyz - centroid) ** 2, axis=-1)
        dist = jnp.minimum(dist, d)
        nxt = jnp.argmax(dist, axis=-1).astype(jnp.int32)
        return (dist, nxt), farthest
    init = (jnp.full((B, N), 1e10, dtype=xyz.dtype), jnp.zeros((B,), dtype=jnp.int32))
    (_, _), idxs = jax.lax.scan(step, init, None, length=s)
    return jnp.transpose(idxs)


def _norm_j(center, knn):
    c = center[:, :, None, :]
    std = jnp.clip(jnp.std(knn - c, axis=(0, 1, 3), keepdims=True, ddof=1), 1e-05, None)
    return (knn - c) / std


def _embed_j(x):
    B = x.shape[0]
    global_std = jnp.mean(jnp.std(x.reshape(B, -1, _IN_DIM), axis=1, ddof=1))
    sigma = _SIGMA * (1.0 + global_std)
    blend = jax.nn.sigmoid((global_std - _BASELINE) * _SCALING)
    fv = jnp.asarray(_FEAT_VAL)
    embeds = []
    for i in range(_IN_DIM):
        tmp = x[..., i:i + 1] - fv
        rbf = jnp.exp(-0.5 * (tmp / (sigma + _EPS)) ** 2)
        cosine = jnp.cos(tmp / (sigma + _EPS))
        embeds.append(blend * rbf + (1.0 - blend) * cosine)
    pe = jnp.concatenate(embeds, axis=-1)
    return jnp.take(pe, jnp.asarray(_OUT_IDX), axis=-1)


def _erf_approx(x):
    ax = jnp.abs(x)
    t = 1.0 / (1.0 + 0.3275911 * ax)
    poly = t * (0.254829592 + t * (-0.284496736 + t * (1.421413741
        + t * (-1.453152027 + t * 1.061405429))))
    y = 1.0 - poly * jnp.exp(-ax * ax)
    return jnp.sign(x) * y


def _gelu_exact(x):
    return 0.5 * x * (1.0 + _erf_approx(x * np.float32(1.0 / math.sqrt(2.0))))


def _gelu_body(x_ref, o_ref):
    o_ref[...] = _gelu_exact(x_ref[...])


def kernel(xyz, feat):
    fps_idx = _fps_j(jax.lax.stop_gradient(xyz), _S)
    xyz_sampled = _idx_pts(xyz, fps_idx)
    feat_sampled = _idx_pts(feat, fps_idx)
    sqr = _sqdist(xyz_sampled, xyz)
    _, idx_knn = jax.lax.top_k(-sqr, _K)
    xyz_knn = _idx_pts(xyz, idx_knn)
    feat_knn = _idx_pts(feat, idx_knn)
    xyz_knn = _norm_j(xyz_sampled, xyz_knn)
    feat_knn = _norm_j(feat_sampled, feat_knn)
    b, s, k, d = feat_knn.shape
    feat_knn = jnp.concatenate(
        [feat_knn, jnp.broadcast_to(feat_sampled[:, :, None, :], (b, s, k, d))], axis=-1)
    pe = _embed_j(xyz_knn)
    w = (feat_knn + pe) * pe
    agg = jnp.mean(w, axis=-2) + jnp.max(w, axis=-2)
    return pl.pallas_call(
        _gelu_body,
        out_shape=jax.ShapeDtypeStruct(agg.shape, agg.dtype),
    )(agg)


# speedup vs baseline: 8.9811x; 8.9811x over previous
import functools
import math

import jax
import jax.numpy as jnp
import numpy as np
from jax import lax
from jax.experimental import pallas as pl
from jax.experimental.pallas import tpu as pltpu
from jax.experimental.pallas import tpu_sc as plsc

_B = 8
_N = 4096
_S = 1024
_K = 32
_D = 128
_W = 256
_XO = 128
_OUT_DIM = 256
_IN_DIM = 3
_SIGMA = 0.26
_BASELINE = 0.1
_SCALING = 10.0
_EPS = 1e-06
_PAD = 16

_fd = math.ceil(_OUT_DIM / _IN_DIM)
_FEAT_NUM = _fd * _IN_DIM
_OUT_IDX_NP = np.linspace(0, _FEAT_NUM - 1, _OUT_DIM).astype(np.int32)
_FEAT_VAL_NP = np.linspace(-1.0, 1.0, _fd + 2)[1:-1].astype(np.float32)
_SELID_NP = (_OUT_IDX_NP // _fd).astype(np.int32).reshape(1, _OUT_DIM)
_FVSEL_NP = _FEAT_VAL_NP[_OUT_IDX_NP % _fd].astype(np.float32).reshape(1, _OUT_DIM)


def _fps_body(xs_ref, ys_ref, zs_ref, idx_ref):
    xs = xs_ref[...]
    ys = ys_ref[...]
    zs = zs_ref[...]
    lane = lax.broadcasted_iota(jnp.int32, (_B, _N), 1)
    lane_s = lax.broadcasted_iota(jnp.int32, (_B, _S), 1)
    rowoff = lax.broadcasted_iota(jnp.int32, (_B, _S), 0) * _N

    def step(t, carry):
        dist, far, acc = carry
        acc = jnp.where(lane_s == t, jnp.broadcast_to(far, (_B, _S)), acc)
        oh = lane == far
        cx = jnp.sum(jnp.where(oh, xs, 0.0), axis=1, keepdims=True)
        cy = jnp.sum(jnp.where(oh, ys, 0.0), axis=1, keepdims=True)
        cz = jnp.sum(jnp.where(oh, zs, 0.0), axis=1, keepdims=True)
        dx = xs - cx
        dy = ys - cy
        dz = zs - cz
        d = (dx * dx + dy * dy) + dz * dz
        dist = jnp.minimum(dist, d)
        m = jnp.max(dist, axis=1, keepdims=True)
        far = jnp.min(jnp.where(dist == m, lane, _N), axis=1, keepdims=True)
        return dist, far, acc

    dist0 = xs * 0.0 + 1e10
    far0 = (xs[:, 0:1] * 0.0).astype(jnp.int32)
    acc0 = (xs[:, :_S] * 0.0).astype(jnp.int32)
    _, _, acc = lax.fori_loop(0, _S, step, (dist0, far0, acc0))
    idx_ref[...] = acc + rowoff


def _fps_call(xs, ys, zs):
    return pl.pallas_call(
        _fps_body,
        out_shape=jax.ShapeDtypeStruct((_B, _S), jnp.int32),
    )(xs, ys, zs)


_TQ = 256


def _knn_body(samp_ref, pt_ref, idx_ref):
    b = pl.program_id(0)
    q = samp_ref[:, _XO:_XO + _PAD]
    p = pt_ref[0]
    mm = lax.dot_general(q, p, (((1,), (0,)), ((), ())),
                         preferred_element_type=jnp.float32)
    qx = q[:, 0:1]
    qy = q[:, 1:2]
    qz = q[:, 2:3]
    qn = (qx * qx + qy * qy) + qz * qz
    px = p[0:1, :]
    py = p[1:2, :]
    pz = p[2:3, :]
    pn = (px * px + py * py) + pz * pz
    d = (-2.0 * mm + qn) + pn

    lane = lax.broadcasted_iota(jnp.int32, (_TQ, _N), 1)
    lane_k = lax.broadcasted_iota(jnp.int32, (_TQ, _K), 1)
    boff = b * _N
    acc = jnp.zeros((_TQ, _K), jnp.int32)
    for k in range(_K):
        m = jnp.min(d, axis=1, keepdims=True)
        sel = d == m
        il = jnp.min(jnp.where(sel, lane, _N), axis=1, keepdims=True)
        acc = jnp.where(lane_k == k, jnp.broadcast_to(il + boff, (_TQ, _K)), acc)
        d = jnp.where(lane == il, jnp.inf, d)
    idx_ref[0] = jnp.transpose(acc)


def _knn_call(samp, pt):
    grid = (_B, _S // _TQ)
    return pl.pallas_call(
        _knn_body,
        grid=grid,
        in_specs=[
            pl.BlockSpec((_TQ, _W), lambda b, st: (b * (_S // _TQ) + st, 0)),
            pl.BlockSpec((1, _PAD, _N), lambda b, st: (b, 0, 0)),
        ],
        out_specs=pl.BlockSpec((1, _K, _TQ), lambda b, st: (b, 0, st)),
        out_shape=jax.ShapeDtypeStruct((_B, _K, _S), jnp.int32),
    )(samp, pt)


_NC = 2
_NS = 16
_NW = _NC * _NS


def _sc_gather_call(tbl, idx_flat):
    n = idx_flat.shape[0]
    per_w = n // _NW
    chunks = per_w // 128
    mesh = plsc.VectorSubcoreMesh(core_axis_name="c", subcore_axis_name="s")

    @functools.partial(
        pl.kernel,
        out_type=jax.ShapeDtypeStruct((n, _W), jnp.float32),
        mesh=mesh,
        scratch_types=[
            pltpu.VMEM((128,), jnp.int32),
            pltpu.VMEM((128, _W), jnp.float32),
            pltpu.SemaphoreType.DMA,
        ],
    )
    def k(tbl_hbm, idx_hbm, out_hbm, idx_v, rows_v, sem):
        wid = lax.axis_index("s") * _NC + lax.axis_index("c")

        def chunk(c, _):
            base = wid * per_w + c * 128
            pltpu.sync_copy(idx_hbm.at[pl.ds(base, 128)], idx_v)
            pltpu.async_copy(tbl_hbm.at[idx_v], rows_v, sem).wait()
            pltpu.sync_copy(rows_v, out_hbm.at[pl.ds(base, 128)])
            return 0

        lax.fori_loop(0, chunks, chunk, 0)

    return k(tbl, idx_flat)


_TS = 256


def _stats_body(rk_ref, rs_ref, a_ref, qx_ref, sf_ref, qf_ref):
    st = pl.program_id(1)
    rk = rk_ref[0]
    rs = rs_ref[...]
    xk = rk[:, :, _XO:_XO + _PAD]
    xs = rs[:, _XO:_XO + _PAD]
    fk = rk[:, :, :_D]
    fs = rs[:, :_D]

    xd = xk - xs[None]
    a_part = jnp.sum(xd, axis=1)
    q_part = jnp.sum(xd * xd, axis=1)
    fdiff = fk - fs[None]
    sf_part = jnp.sum(jnp.sum(fdiff, axis=2), axis=1)
    qf_part = jnp.sum(jnp.sum(fdiff * fdiff, axis=2), axis=1)

    @pl.when(st == 0)
    def _():
        a_ref[...] = jnp.zeros_like(a_ref)
        qx_ref[...] = jnp.zeros_like(qx_ref)
        sf_ref[...] = jnp.zeros_like(sf_ref)
        qf_ref[...] = jnp.zeros_like(qf_ref)

    a_ref[...] += a_part[None]
    qx_ref[...] += q_part[None]
    sf_ref[...] += sf_part.reshape(1, 1, _K)
    qf_ref[...] += qf_part.reshape(1, 1, _K)


def _stats_call(rk4, samp):
    grid = (_B, _S // _TS)
    return pl.pallas_call(
        _stats_body,
        grid=grid,
        in_specs=[
            pl.BlockSpec((1, _K, _TS, _W), lambda b, st: (b, 0, st, 0)),
            pl.BlockSpec((_TS, _W), lambda b, st: (b * (_S // _TS) + st, 0)),
        ],
        out_specs=[
            pl.BlockSpec((1, _K, _PAD), lambda b, st: (b, 0, 0)),
            pl.BlockSpec((1, _K, _PAD), lambda b, st: (b, 0, 0)),
            pl.BlockSpec((1, 1, _K), lambda b, st: (b, 0, 0)),
            pl.BlockSpec((1, 1, _K), lambda b, st: (b, 0, 0)),
        ],
        out_shape=[
            jax.ShapeDtypeStruct((_B, _K, _PAD), jnp.float32),
            jax.ShapeDtypeStruct((_B, _K, _PAD), jnp.float32),
            jax.ShapeDtypeStruct((_B, 1, _K), jnp.float32),
            jax.ShapeDtypeStruct((_B, 1, _K), jnp.float32),
        ],
    )(rk4, samp)


_TM = 128


def _erf_approx(x):
    ax = jnp.abs(x)
    t = 1.0 / (1.0 + 0.3275911 * ax)
    poly = t * (0.254829592 + t * (-0.284496736 + t * (1.421413741
        + t * (-1.453152027 + t * 1.061405429))))
    y = 1.0 - poly * jnp.exp(-ax * ax)
    return jnp.sign(x) * y


def _gelu_exact(x):
    return 0.5 * x * (1.0 + _erf_approx(x * np.float32(1.0 / math.sqrt(2.0))))


def _main_body(rk_ref, rs_ref, a_ref, qx_ref, sf_ref, qf_ref,
               fv_ref, sid_ref, o_ref):
    a = a_ref[...]
    qx = qx_ref[...]
    nx = np.float32(_B * _S * _IN_DIM)
    sum_a = jnp.sum(jnp.sum(a, axis=0), axis=1, keepdims=True)
    sum_q = jnp.sum(jnp.sum(qx, axis=0), axis=1, keepdims=True)
    var_x = (sum_q - sum_a * sum_a / nx) / (nx - 1.0)
    std_x = jnp.maximum(jnp.sqrt(jnp.maximum(var_x, 0.0)), 1e-05)
    inv_x = 1.0 / std_x

    sk = np.float32(_S * _K)
    an = a * inv_x[None]
    qn = qx * (inv_x * inv_x)[None]
    sum_bd = jnp.sum(an, axis=1)
    ssq_bd = jnp.sum(qn, axis=1)
    var_bd = (ssq_bd - sum_bd * sum_bd / sk) / (sk - 1.0)
    gs = jnp.sum(jnp.sqrt(jnp.maximum(var_bd, 0.0))) / np.float32(_B * _IN_DIM)
    sigma = _SIGMA * (1.0 + gs)
    r = 1.0 / (sigma + _EPS)
    blend = 1.0 / (1.0 + jnp.exp(-(gs - _BASELINE) * _SCALING))
    one_m_blend = 1.0 - blend

    nf = np.float32(_B * _S * _D)
    sum_f = jnp.sum(sf_ref[...], axis=0)
    sum_qf = jnp.sum(qf_ref[...], axis=0)
    var_f = (sum_qf - sum_f * sum_f / nf) / (nf - 1.0)
    std_f = jnp.maximum(jnp.sqrt(jnp.maximum(var_f, 0.0)), 1e-05)
    inv_f = 1.0 / std_f

    fv = fv_ref[...]
    sid = sid_ref[...]
    rs = rs_ref[...]
    fs = rs[:, :_D]
    xs = rs[:, _XO:_XO + _PAD]

    neg_inf = np.float32(-np.inf)
    for half in range(2):
        fvh = fv[:, half * _D:(half + 1) * _D]
        sidh = sid[:, half * _D:(half + 1) * _D]
        s_acc = jnp.zeros((_TM, _D), jnp.float32)
        m_acc = jnp.full((_TM, _D), neg_inf, jnp.float32)
        for k in range(_K):
            row = rk_ref[0, k]
            xn = (row[:, _XO:_XO + _PAD] - xs) * inv_x[k:k + 1, 0:1]
            x0 = xn[:, 0:1]
            x1 = xn[:, 1:2]
            x2 = xn[:, 2:3]
            xsel = jnp.where(sidh == 0, x0, jnp.where(sidh == 1, x1, x2))
            t = (xsel - fvh) * r
            pe = blend * jnp.exp(-0.5 * (t * t)) + one_m_blend * jnp.cos(t)
            if half == 0:
                fc = (row[:, :_D] - fs) * inv_f[0:1, k:k + 1]
            else:
                fc = fs
            w = (fc + pe) * pe
            s_acc = s_acc + w
            m_acc = jnp.maximum(m_acc, w)
        agg = s_acc * np.float32(1.0 / _K) + m_acc
        o_ref[0, :, half * _D:(half + 1) * _D] = _gelu_exact(agg)


def _main_call(rk4, samp, a, qx, sf, qf):
    grid = (_B, _S // _TM)
    fv = jnp.asarray(_FVSEL_NP)
    sid = jnp.asarray(_SELID_NP)
    return pl.pallas_call(
        _main_body,
        grid=grid,
        in_specs=[
            pl.BlockSpec((1, _K, _TM, _W), lambda b, st: (b, 0, st, 0)),
            pl.BlockSpec((_TM, _W), lambda b, st: (b * (_S // _TM) + st, 0)),
            pl.BlockSpec((_B, _K, _PAD), lambda b, st: (0, 0, 0)),
            pl.BlockSpec((_B, _K, _PAD), lambda b, st: (0, 0, 0)),
            pl.BlockSpec((_B, 1, _K), lambda b, st: (0, 0, 0)),
            pl.BlockSpec((_B, 1, _K), lambda b, st: (0, 0, 0)),
            pl.BlockSpec((1, _OUT_DIM), lambda b, st: (0, 0)),
            pl.BlockSpec((1, _OUT_DIM), lambda b, st: (0, 0)),
        ],
        out_specs=pl.BlockSpec((1, _TM, _OUT_DIM), lambda b, st: (b, st, 0)),
        out_shape=jax.ShapeDtypeStruct((_B, _S, _OUT_DIM), jnp.float32),
    )(rk4, samp, a, qx, sf, qf, fv, sid)


def kernel(xyz, feat):
    xs = xyz[:, :, 0]
    ys = xyz[:, :, 1]
    zs = xyz[:, :, 2]
    xyzp = jnp.pad(xyz, ((0, 0), (0, 0), (0, _PAD - _IN_DIM)))
    pt = jnp.transpose(xyzp, (0, 2, 1))
    tbl = jnp.concatenate(
        [feat, jnp.pad(xyz, ((0, 0), (0, 0), (0, _W - _D - _IN_DIM)))],
        axis=-1).reshape(_B * _N, _W)

    fps_idx = _fps_call(xs, ys, zs)
    samp = _sc_gather_call(tbl, fps_idx.reshape(-1))
    idx_t = _knn_call(samp, pt)
    rows_k = _sc_gather_call(tbl, idx_t.reshape(-1))
    rk4 = rows_k.reshape(_B, _K, _S, _W)
    a, qx, sf, qf = _stats_call(rk4, samp)
    return _main_call(rk4, samp, a, qx, sf, qf)


# speedup vs baseline: 9.8132x; 1.0926x over previous
import functools
import math

import jax
import jax.numpy as jnp
import numpy as np
from jax import lax
from jax.experimental import pallas as pl
from jax.experimental.pallas import tpu as pltpu
from jax.experimental.pallas import tpu_sc as plsc

_B = 8
_N = 4096
_S = 1024
_K = 32
_D = 128
_W = 256
_XO = 128
_OUT_DIM = 256
_IN_DIM = 3
_SIGMA = 0.26
_BASELINE = 0.1
_SCALING = 10.0
_EPS = 1e-06
_PAD = 16

_fd = math.ceil(_OUT_DIM / _IN_DIM)
_FEAT_NUM = _fd * _IN_DIM
_OUT_IDX_NP = np.linspace(0, _FEAT_NUM - 1, _OUT_DIM).astype(np.int32)
_FEAT_VAL_NP = np.linspace(-1.0, 1.0, _fd + 2)[1:-1].astype(np.float32)
_SELID_NP = (_OUT_IDX_NP // _fd).astype(np.int32).reshape(1, _OUT_DIM)
_FVSEL_NP = _FEAT_VAL_NP[_OUT_IDX_NP % _fd].astype(np.float32).reshape(1, _OUT_DIM)


def _fps_body(xs_ref, ys_ref, zs_ref, idx_ref):
    xs = xs_ref[...]
    ys = ys_ref[...]
    zs = zs_ref[...]
    lane = lax.broadcasted_iota(jnp.int32, (_B, _N), 1)
    lane_s = lax.broadcasted_iota(jnp.int32, (_B, _S), 1)
    rowoff = lax.broadcasted_iota(jnp.int32, (_B, _S), 0) * _N

    def step(t, carry):
        dist, far, acc = carry
        acc = jnp.where(lane_s == t, jnp.broadcast_to(far, (_B, _S)), acc)
        oh = lane == far
        cx = jnp.sum(jnp.where(oh, xs, 0.0), axis=1, keepdims=True)
        cy = jnp.sum(jnp.where(oh, ys, 0.0), axis=1, keepdims=True)
        cz = jnp.sum(jnp.where(oh, zs, 0.0), axis=1, keepdims=True)
        dx = xs - cx
        dy = ys - cy
        dz = zs - cz
        d = (dx * dx + dy * dy) + dz * dz
        dist = jnp.minimum(dist, d)
        far = jnp.argmax(dist, axis=1).astype(jnp.int32)[:, None]
        return dist, far, acc

    dist0 = xs * 0.0 + 1e10
    far0 = (xs[:, 0:1] * 0.0).astype(jnp.int32)
    acc0 = (xs[:, :_S] * 0.0).astype(jnp.int32)
    _, _, acc = lax.fori_loop(0, _S, step, (dist0, far0, acc0))
    idx_ref[...] = acc + rowoff


def _fps_call(xs, ys, zs):
    return pl.pallas_call(
        _fps_body,
        out_shape=jax.ShapeDtypeStruct((_B, _S), jnp.int32),
    )(xs, ys, zs)


_TQ = 256


def _knn_body(samp_ref, pt_ref, idx_ref):
    b = pl.program_id(0)
    q = samp_ref[:, _XO:_XO + _PAD]
    p = pt_ref[0]
    mm = lax.dot_general(q, p, (((1,), (0,)), ((), ())),
                         preferred_element_type=jnp.float32)
    qx = q[:, 0:1]
    qy = q[:, 1:2]
    qz = q[:, 2:3]
    qn = (qx * qx + qy * qy) + qz * qz
    px = p[0:1, :]
    py = p[1:2, :]
    pz = p[2:3, :]
    pn = (px * px + py * py) + pz * pz
    d = (-2.0 * mm + qn) + pn

    lane = lax.broadcasted_iota(jnp.int32, (_TQ, _N), 1)
    lane_k = lax.broadcasted_iota(jnp.int32, (_TQ, _K), 1)
    boff = b * _N
    acc = jnp.zeros((_TQ, _K), jnp.int32)
    for k in range(_K):
        il = jnp.argmin(d, axis=1).astype(jnp.int32)[:, None]
        acc = jnp.where(lane_k == k, jnp.broadcast_to(il + boff, (_TQ, _K)), acc)
        d = jnp.where(lane == il, jnp.inf, d)
    idx_ref[0] = jnp.transpose(acc)


def _knn_call(samp, pt):
    grid = (_B, _S // _TQ)
    return pl.pallas_call(
        _knn_body,
        grid=grid,
        in_specs=[
            pl.BlockSpec((_TQ, _W), lambda b, st: (b * (_S // _TQ) + st, 0)),
            pl.BlockSpec((1, _PAD, _N), lambda b, st: (b, 0, 0)),
        ],
        out_specs=pl.BlockSpec((1, _K, _TQ), lambda b, st: (b, 0, st)),
        out_shape=jax.ShapeDtypeStruct((_B, _K, _S), jnp.int32),
    )(samp, pt)


_NC = 2
_NS = 16
_NW = _NC * _NS


def _sc_gather_call(tbl, idx_flat):
    n = idx_flat.shape[0]
    per_w = n // _NW
    chunks = per_w // 128
    mesh = plsc.VectorSubcoreMesh(core_axis_name="c", subcore_axis_name="s")

    @functools.partial(
        pl.kernel,
        out_type=jax.ShapeDtypeStruct((n, _W), jnp.float32),
        mesh=mesh,
        scratch_types=[
            pltpu.VMEM((128,), jnp.int32),
            pltpu.VMEM((128, _W), jnp.float32),
            pltpu.SemaphoreType.DMA,
        ],
    )
    def k(tbl_hbm, idx_hbm, out_hbm, idx_v, rows_v, sem):
        wid = lax.axis_index("s") * _NC + lax.axis_index("c")

        def chunk(c, _):
            base = wid * per_w + c * 128
            pltpu.sync_copy(idx_hbm.at[pl.ds(base, 128)], idx_v)
            pltpu.async_copy(tbl_hbm.at[idx_v], rows_v, sem).wait()
            pltpu.sync_copy(rows_v, out_hbm.at[pl.ds(base, 128)])
            return 0

        lax.fori_loop(0, chunks, chunk, 0)

    return k(tbl, idx_flat)


_TS = 256


def _stats_body(rk_ref, rs_ref, a_ref, qx_ref, sf_ref, qf_ref):
    st = pl.program_id(1)
    rk = rk_ref[0]
    rs = rs_ref[...]
    xk = rk[:, :, _XO:_XO + _PAD]
    xs = rs[:, _XO:_XO + _PAD]
    fk = rk[:, :, :_D]
    fs = rs[:, :_D]

    xd = xk - xs[None]
    a_part = jnp.sum(xd, axis=1)
    q_part = jnp.sum(xd * xd, axis=1)
    fdiff = fk - fs[None]
    sf_part = jnp.sum(jnp.sum(fdiff, axis=2), axis=1)
    qf_part = jnp.sum(jnp.sum(fdiff * fdiff, axis=2), axis=1)

    @pl.when(st == 0)
    def _():
        a_ref[...] = jnp.zeros_like(a_ref)
        qx_ref[...] = jnp.zeros_like(qx_ref)
        sf_ref[...] = jnp.zeros_like(sf_ref)
        qf_ref[...] = jnp.zeros_like(qf_ref)

    a_ref[...] += a_part[None]
    qx_ref[...] += q_part[None]
    sf_ref[...] += sf_part.reshape(1, 1, _K)
    qf_ref[...] += qf_part.reshape(1, 1, _K)


def _stats_call(rk4, samp):
    grid = (_B, _S // _TS)
    return pl.pallas_call(
        _stats_body,
        grid=grid,
        in_specs=[
            pl.BlockSpec((1, _K, _TS, _W), lambda b, st: (b, 0, st, 0)),
            pl.BlockSpec((_TS, _W), lambda b, st: (b * (_S // _TS) + st, 0)),
        ],
        out_specs=[
            pl.BlockSpec((1, _K, _PAD), lambda b, st: (b, 0, 0)),
            pl.BlockSpec((1, _K, _PAD), lambda b, st: (b, 0, 0)),
            pl.BlockSpec((1, 1, _K), lambda b, st: (b, 0, 0)),
            pl.BlockSpec((1, 1, _K), lambda b, st: (b, 0, 0)),
        ],
        out_shape=[
            jax.ShapeDtypeStruct((_B, _K, _PAD), jnp.float32),
            jax.ShapeDtypeStruct((_B, _K, _PAD), jnp.float32),
            jax.ShapeDtypeStruct((_B, 1, _K), jnp.float32),
            jax.ShapeDtypeStruct((_B, 1, _K), jnp.float32),
        ],
    )(rk4, samp)


_TM = 128


def _erf_approx(x):
    ax = jnp.abs(x)
    t = 1.0 / (1.0 + 0.3275911 * ax)
    poly = t * (0.254829592 + t * (-0.284496736 + t * (1.421413741
        + t * (-1.453152027 + t * 1.061405429))))
    y = 1.0 - poly * jnp.exp(-ax * ax)
    return jnp.sign(x) * y


def _gelu_exact(x):
    return 0.5 * x * (1.0 + _erf_approx(x * np.float32(1.0 / math.sqrt(2.0))))


def _main_body(rk_ref, rs_ref, a_ref, qx_ref, sf_ref, qf_ref,
               fv_ref, sid_ref, o_ref):
    a = a_ref[...]
    qx = qx_ref[...]
    nx = np.float32(_B * _S * _IN_DIM)
    sum_a = jnp.sum(jnp.sum(a, axis=0), axis=1, keepdims=True)
    sum_q = jnp.sum(jnp.sum(qx, axis=0), axis=1, keepdims=True)
    var_x = (sum_q - sum_a * sum_a / nx) / (nx - 1.0)
    std_x = jnp.maximum(jnp.sqrt(jnp.maximum(var_x, 0.0)), 1e-05)
    inv_x = 1.0 / std_x

    sk = np.float32(_S * _K)
    an = a * inv_x[None]
    qn = qx * (inv_x * inv_x)[None]
    sum_bd = jnp.sum(an, axis=1)
    ssq_bd = jnp.sum(qn, axis=1)
    var_bd = (ssq_bd - sum_bd * sum_bd / sk) / (sk - 1.0)
    gs = jnp.sum(jnp.sqrt(jnp.maximum(var_bd, 0.0))) / np.float32(_B * _IN_DIM)
    sigma = _SIGMA * (1.0 + gs)
    r = 1.0 / (sigma + _EPS)
    blend = 1.0 / (1.0 + jnp.exp(-(gs - _BASELINE) * _SCALING))
    one_m_blend = 1.0 - blend

    nf = np.float32(_B * _S * _D)
    sum_f = jnp.sum(sf_ref[...], axis=0)
    sum_qf = jnp.sum(qf_ref[...], axis=0)
    var_f = (sum_qf - sum_f * sum_f / nf) / (nf - 1.0)
    std_f = jnp.maximum(jnp.sqrt(jnp.maximum(var_f, 0.0)), 1e-05)
    inv_f = 1.0 / std_f

    fv = fv_ref[...]
    sid = sid_ref[...]
    rs = rs_ref[...]
    fs = rs[:, :_D]
    xs = rs[:, _XO:_XO + _PAD]

    neg_inf = np.float32(-np.inf)
    for half in range(2):
        fvh = fv[:, half * _D:(half + 1) * _D]
        sidh = sid[:, half * _D:(half + 1) * _D]
        s_acc = jnp.zeros((_TM, _D), jnp.float32)
        m_acc = jnp.full((_TM, _D), neg_inf, jnp.float32)
        for k in range(_K):
            row = rk_ref[0, k]
            xn = (row[:, _XO:_XO + _PAD] - xs) * inv_x[k:k + 1, 0:1]
            x0 = xn[:, 0:1]
            x1 = xn[:, 1:2]
            x2 = xn[:, 2:3]
            xsel = jnp.where(sidh == 0, x0, jnp.where(sidh == 1, x1, x2))
            t = (xsel - fvh) * r
            pe = blend * jnp.exp(-0.5 * (t * t)) + one_m_blend * jnp.cos(t)
            if half == 0:
                fc = (row[:, :_D] - fs) * inv_f[0:1, k:k + 1]
            else:
                fc = fs
            w = (fc + pe) * pe
            s_acc = s_acc + w
            m_acc = jnp.maximum(m_acc, w)
        agg = s_acc * np.float32(1.0 / _K) + m_acc
        o_ref[0, :, half * _D:(half + 1) * _D] = _gelu_exact(agg)


def _main_call(rk4, samp, a, qx, sf, qf):
    grid = (_B, _S // _TM)
    fv = jnp.asarray(_FVSEL_NP)
    sid = jnp.asarray(_SELID_NP)
    return pl.pallas_call(
        _main_body,
        grid=grid,
        in_specs=[
            pl.BlockSpec((1, _K, _TM, _W), lambda b, st: (b, 0, st, 0)),
            pl.BlockSpec((_TM, _W), lambda b, st: (b * (_S // _TM) + st, 0)),
            pl.BlockSpec((_B, _K, _PAD), lambda b, st: (0, 0, 0)),
            pl.BlockSpec((_B, _K, _PAD), lambda b, st: (0, 0, 0)),
            pl.BlockSpec((_B, 1, _K), lambda b, st: (0, 0, 0)),
            pl.BlockSpec((_B, 1, _K), lambda b, st: (0, 0, 0)),
            pl.BlockSpec((1, _OUT_DIM), lambda b, st: (0, 0)),
            pl.BlockSpec((1, _OUT_DIM), lambda b, st: (0, 0)),
        ],
        out_specs=pl.BlockSpec((1, _TM, _OUT_DIM), lambda b, st: (b, st, 0)),
        out_shape=jax.ShapeDtypeStruct((_B, _S, _OUT_DIM), jnp.float32),
    )(rk4, samp, a, qx, sf, qf, fv, sid)


def kernel(xyz, feat):
    xs = xyz[:, :, 0]
    ys = xyz[:, :, 1]
    zs = xyz[:, :, 2]
    xyzp = jnp.pad(xyz, ((0, 0), (0, 0), (0, _PAD - _IN_DIM)))
    pt = jnp.transpose(xyzp, (0, 2, 1))
    tbl = jnp.concatenate(
        [feat, jnp.pad(xyz, ((0, 0), (0, 0), (0, _W - _D - _IN_DIM)))],
        axis=-1).reshape(_B * _N, _W)

    fps_idx = _fps_call(xs, ys, zs)
    samp = _sc_gather_call(tbl, fps_idx.reshape(-1))
    idx_t = _knn_call(samp, pt)
    samp = _sc_gather_call(tbl, fps_idx.reshape(-1))
    idx_t = _knn_call(samp, pt)
    rows_k = _sc_gather_call(tbl, idx_t.reshape(-1))
    rk4 = rows_k.reshape(_B, _K, _S, _W)
    a, qx, sf, qf = _stats_call(rk4, samp)
    return _main_call(rk4, samp, a, qx, sf, qf)


# speedup vs baseline: 10.9599x; 1.1169x over previous
import functools
import math

import jax
import jax.numpy as jnp
import numpy as np
from jax import lax
from jax.experimental import pallas as pl
from jax.experimental.pallas import tpu as pltpu
from jax.experimental.pallas import tpu_sc as plsc

_B = 8
_N = 4096
_S = 1024
_K = 32
_D = 128
_W = 256
_XO = 128
_OUT_DIM = 256
_IN_DIM = 3
_SIGMA = 0.26
_BASELINE = 0.1
_SCALING = 10.0
_EPS = 1e-06
_PAD = 16

_fd = math.ceil(_OUT_DIM / _IN_DIM)
_FEAT_NUM = _fd * _IN_DIM
_OUT_IDX_NP = np.linspace(0, _FEAT_NUM - 1, _OUT_DIM).astype(np.int32)
_FEAT_VAL_NP = np.linspace(-1.0, 1.0, _fd + 2)[1:-1].astype(np.float32)
_SELID_NP = (_OUT_IDX_NP // _fd).astype(np.int32).reshape(1, _OUT_DIM)
_FVSEL_NP = _FEAT_VAL_NP[_OUT_IDX_NP % _fd].astype(np.float32).reshape(1, _OUT_DIM)


def _fps_body(xs_ref, ys_ref, zs_ref, idx_ref):
    xs = xs_ref[...]
    ys = ys_ref[...]
    zs = zs_ref[...]
    lane = lax.broadcasted_iota(jnp.int32, (_B, _N), 1)
    lane_s = lax.broadcasted_iota(jnp.int32, (_B, _S), 1)
    rowoff = lax.broadcasted_iota(jnp.int32, (_B, _S), 0) * _N

    def step(t, carry):
        dist, far, acc = carry
        acc = jnp.where(lane_s == t, jnp.broadcast_to(far, (_B, _S)), acc)
        oh = lane == far
        cx = jnp.sum(jnp.where(oh, xs, 0.0), axis=1, keepdims=True)
        cy = jnp.sum(jnp.where(oh, ys, 0.0), axis=1, keepdims=True)
        cz = jnp.sum(jnp.where(oh, zs, 0.0), axis=1, keepdims=True)
        dx = xs - cx
        dy = ys - cy
        dz = zs - cz
        d = (dx * dx + dy * dy) + dz * dz
        dist = jnp.minimum(dist, d)
        far = jnp.argmax(dist, axis=1).astype(jnp.int32)[:, None]
        return dist, far, acc

    dist0 = xs * 0.0 + 1e10
    far0 = (xs[:, 0:1] * 0.0).astype(jnp.int32)
    acc0 = (xs[:, :_S] * 0.0).astype(jnp.int32)
    _, _, acc = lax.fori_loop(0, _S, step, (dist0, far0, acc0))
    idx_ref[...] = acc + rowoff


def _fps_call(xs, ys, zs):
    return pl.pallas_call(
        _fps_body,
        out_shape=jax.ShapeDtypeStruct((_B, _S), jnp.int32),
    )(xs, ys, zs)


_TQ = 256


def _knn_body(samp_ref, pt_ref, idx_ref):
    b = pl.program_id(0)
    q = samp_ref[:, _XO:_XO + _PAD]
    p = pt_ref[0]
    mm = lax.dot_general(q, p, (((1,), (0,)), ((), ())),
                         preferred_element_type=jnp.float32)
    qx = q[:, 0:1]
    qy = q[:, 1:2]
    qz = q[:, 2:3]
    qn = (qx * qx + qy * qy) + qz * qz
    px = p[0:1, :]
    py = p[1:2, :]
    pz = p[2:3, :]
    pn = (px * px + py * py) + pz * pz
    d = (-2.0 * mm + qn) + pn

    lane = lax.broadcasted_iota(jnp.int32, (_TQ, _N), 1)
    lane_k = lax.broadcasted_iota(jnp.int32, (_TQ, _K), 1)
    boff = b * _N
    acc = jnp.zeros((_TQ, _K), jnp.int32)
    for k in range(_K):
        il = jnp.argmin(d, axis=1).astype(jnp.int32)[:, None]
        acc = jnp.where(lane_k == k, jnp.broadcast_to(il + boff, (_TQ, _K)), acc)
        d = jnp.where(lane == il, jnp.inf, d)
    idx_ref[0] = jnp.transpose(acc)


def _knn_call(samp, pt):
    grid = (_B, _S // _TQ)
    return pl.pallas_call(
        _knn_body,
        grid=grid,
        in_specs=[
            pl.BlockSpec((_TQ, _W), lambda b, st: (b * (_S // _TQ) + st, 0)),
            pl.BlockSpec((1, _PAD, _N), lambda b, st: (b, 0, 0)),
        ],
        out_specs=pl.BlockSpec((1, _K, _TQ), lambda b, st: (b, 0, st)),
        out_shape=jax.ShapeDtypeStruct((_B, _K, _S), jnp.int32),
    )(samp, pt)


_NC = 2
_NS = 16
_NW = _NC * _NS


def _sc_gather_call(tbl, idx_flat):
    n = idx_flat.shape[0]
    per_w = n // _NW
    chunks = per_w // 128
    mesh = plsc.VectorSubcoreMesh(core_axis_name="c", subcore_axis_name="s")

    @functools.partial(
        pl.kernel,
        out_type=jax.ShapeDtypeStruct((n, _W), jnp.float32),
        mesh=mesh,
        scratch_types=[
            pltpu.VMEM((128,), jnp.int32),
            pltpu.VMEM((128, _W), jnp.float32),
            pltpu.SemaphoreType.DMA,
        ],
    )
    def k(tbl_hbm, idx_hbm, out_hbm, idx_v, rows_v, sem):
        wid = lax.axis_index("s") * _NC + lax.axis_index("c")

        def chunk(c, _):
            base = wid * per_w + c * 128
            pltpu.sync_copy(idx_hbm.at[pl.ds(base, 128)], idx_v)
            pltpu.async_copy(tbl_hbm.at[idx_v], rows_v, sem).wait()
            pltpu.sync_copy(rows_v, out_hbm.at[pl.ds(base, 128)])
            return 0

        lax.fori_loop(0, chunks, chunk, 0)

    return k(tbl, idx_flat)


_TS = 256


def _stats_body(rk_ref, rs_ref, a_ref, qx_ref, sf_ref, qf_ref):
    st = pl.program_id(1)
    rk = rk_ref[0]
    rs = rs_ref[...]
    xk = rk[:, :, _XO:_XO + _PAD]
    xs = rs[:, _XO:_XO + _PAD]
    fk = rk[:, :, :_D]
    fs = rs[:, :_D]

    xd = xk - xs[None]
    a_part = jnp.sum(xd, axis=1)
    q_part = jnp.sum(xd * xd, axis=1)
    fdiff = fk - fs[None]
    sf_part = jnp.sum(jnp.sum(fdiff, axis=2), axis=1)
    qf_part = jnp.sum(jnp.sum(fdiff * fdiff, axis=2), axis=1)

    @pl.when(st == 0)
    def _():
        a_ref[...] = jnp.zeros_like(a_ref)
        qx_ref[...] = jnp.zeros_like(qx_ref)
        sf_ref[...] = jnp.zeros_like(sf_ref)
        qf_ref[...] = jnp.zeros_like(qf_ref)

    a_ref[...] += a_part[None]
    qx_ref[...] += q_part[None]
    sf_ref[...] += sf_part.reshape(1, 1, _K)
    qf_ref[...] += qf_part.reshape(1, 1, _K)


def _stats_call(rk4, samp):
    grid = (_B, _S // _TS)
    return pl.pallas_call(
        _stats_body,
        grid=grid,
        in_specs=[
            pl.BlockSpec((1, _K, _TS, _W), lambda b, st: (b, 0, st, 0)),
            pl.BlockSpec((_TS, _W), lambda b, st: (b * (_S // _TS) + st, 0)),
        ],
        out_specs=[
            pl.BlockSpec((1, _K, _PAD), lambda b, st: (b, 0, 0)),
            pl.BlockSpec((1, _K, _PAD), lambda b, st: (b, 0, 0)),
            pl.BlockSpec((1, 1, _K), lambda b, st: (b, 0, 0)),
            pl.BlockSpec((1, 1, _K), lambda b, st: (b, 0, 0)),
        ],
        out_shape=[
            jax.ShapeDtypeStruct((_B, _K, _PAD), jnp.float32),
            jax.ShapeDtypeStruct((_B, _K, _PAD), jnp.float32),
            jax.ShapeDtypeStruct((_B, 1, _K), jnp.float32),
            jax.ShapeDtypeStruct((_B, 1, _K), jnp.float32),
        ],
    )(rk4, samp)


_TM = 128


def _erf_approx(x):
    ax = jnp.abs(x)
    t = 1.0 / (1.0 + 0.3275911 * ax)
    poly = t * (0.254829592 + t * (-0.284496736 + t * (1.421413741
        + t * (-1.453152027 + t * 1.061405429))))
    y = 1.0 - poly * jnp.exp(-ax * ax)
    return jnp.sign(x) * y


def _gelu_exact(x):
    return 0.5 * x * (1.0 + _erf_approx(x * np.float32(1.0 / math.sqrt(2.0))))


def _main_body(rk_ref, rs_ref, a_ref, qx_ref, sf_ref, qf_ref,
               fv_ref, sid_ref, o_ref):
    a = a_ref[...]
    qx = qx_ref[...]
    nx = np.float32(_B * _S * _IN_DIM)
    sum_a = jnp.sum(jnp.sum(a, axis=0), axis=1, keepdims=True)
    sum_q = jnp.sum(jnp.sum(qx, axis=0), axis=1, keepdims=True)
    var_x = (sum_q - sum_a * sum_a / nx) / (nx - 1.0)
    std_x = jnp.maximum(jnp.sqrt(jnp.maximum(var_x, 0.0)), 1e-05)
    inv_x = 1.0 / std_x

    sk = np.float32(_S * _K)
    an = a * inv_x[None]
    qn = qx * (inv_x * inv_x)[None]
    sum_bd = jnp.sum(an, axis=1)
    ssq_bd = jnp.sum(qn, axis=1)
    var_bd = (ssq_bd - sum_bd * sum_bd / sk) / (sk - 1.0)
    gs = jnp.sum(jnp.sqrt(jnp.maximum(var_bd, 0.0))) / np.float32(_B * _IN_DIM)
    sigma = _SIGMA * (1.0 + gs)
    r = 1.0 / (sigma + _EPS)
    blend = 1.0 / (1.0 + jnp.exp(-(gs - _BASELINE) * _SCALING))
    one_m_blend = 1.0 - blend

    nf = np.float32(_B * _S * _D)
    sum_f = jnp.sum(sf_ref[...], axis=0)
    sum_qf = jnp.sum(qf_ref[...], axis=0)
    var_f = (sum_qf - sum_f * sum_f / nf) / (nf - 1.0)
    std_f = jnp.maximum(jnp.sqrt(jnp.maximum(var_f, 0.0)), 1e-05)
    inv_f = 1.0 / std_f

    fv = fv_ref[...]
    sid = sid_ref[...]
    rs = rs_ref[...]
    fs = rs[:, :_D]
    xs = rs[:, _XO:_XO + _PAD]

    neg_inf = np.float32(-np.inf)
    cfv = one_m_blend * jnp.cos(fv * r)
    sfv = one_m_blend * jnp.sin(fv * r)
    s_acc = [jnp.zeros((_TM, _D), jnp.float32) for _ in range(2)]
    m_acc = [jnp.full((_TM, _D), neg_inf, jnp.float32) for _ in range(2)]
    for k in range(_K):
        row = rk_ref[0, k]
        xn = (row[:, _XO:_XO + _PAD] - xs) * inv_x[k:k + 1, 0:1]
        rx = xn * r
        crx = jnp.cos(rx)
        srx = jnp.sin(rx)
        for half in range(2):
            fvh = fv[:, half * _D:(half + 1) * _D]
            sidh = sid[:, half * _D:(half + 1) * _D]
            xsel = jnp.where(sidh == 0, xn[:, 0:1],
                             jnp.where(sidh == 1, xn[:, 1:2], xn[:, 2:3]))
            csel = jnp.where(sidh == 0, crx[:, 0:1],
                             jnp.where(sidh == 1, crx[:, 1:2], crx[:, 2:3]))
            ssel = jnp.where(sidh == 0, srx[:, 0:1],
                             jnp.where(sidh == 1, srx[:, 1:2], srx[:, 2:3]))
            t = (xsel - fvh) * r
            pe = (blend * jnp.exp(-0.5 * (t * t))
                  + (csel * cfv[:, half * _D:(half + 1) * _D]
                     + ssel * sfv[:, half * _D:(half + 1) * _D]))
            if half == 0:
                fc = (row[:, :_D] - fs) * inv_f[0:1, k:k + 1]
            else:
                fc = fs
            w = (fc + pe) * pe
            s_acc[half] = s_acc[half] + w
            m_acc[half] = jnp.maximum(m_acc[half], w)
    for half in range(2):
        agg = s_acc[half] * np.float32(1.0 / _K) + m_acc[half]
        o_ref[0, :, half * _D:(half + 1) * _D] = _gelu_exact(agg)


def _main_call(rk4, samp, a, qx, sf, qf):
    grid = (_B, _S // _TM)
    fv = jnp.asarray(_FVSEL_NP)
    sid = jnp.asarray(_SELID_NP)
    return pl.pallas_call(
        _main_body,
        grid=grid,
        in_specs=[
            pl.BlockSpec((1, _K, _TM, _W), lambda b, st: (b, 0, st, 0)),
            pl.BlockSpec((_TM, _W), lambda b, st: (b * (_S // _TM) + st, 0)),
            pl.BlockSpec((_B, _K, _PAD), lambda b, st: (0, 0, 0)),
            pl.BlockSpec((_B, _K, _PAD), lambda b, st: (0, 0, 0)),
            pl.BlockSpec((_B, 1, _K), lambda b, st: (0, 0, 0)),
            pl.BlockSpec((_B, 1, _K), lambda b, st: (0, 0, 0)),
            pl.BlockSpec((1, _OUT_DIM), lambda b, st: (0, 0)),
            pl.BlockSpec((1, _OUT_DIM), lambda b, st: (0, 0)),
        ],
        out_specs=pl.BlockSpec((1, _TM, _OUT_DIM), lambda b, st: (b, st, 0)),
        out_shape=jax.ShapeDtypeStruct((_B, _S, _OUT_DIM), jnp.float32),
    )(rk4, samp, a, qx, sf, qf, fv, sid)


def kernel(xyz, feat):
    xs = xyz[:, :, 0]
    ys = xyz[:, :, 1]
    zs = xyz[:, :, 2]
    xyzp = jnp.pad(xyz, ((0, 0), (0, 0), (0, _PAD - _IN_DIM)))
    pt = jnp.transpose(xyzp, (0, 2, 1))
    tbl = jnp.concatenate(
        [feat, jnp.pad(xyz, ((0, 0), (0, 0), (0, _W - _D - _IN_DIM)))],
        axis=-1).reshape(_B * _N, _W)

    fps_idx = _fps_call(xs, ys, zs)
    samp = _sc_gather_call(tbl, fps_idx.reshape(-1))
    idx_t = _knn_call(samp, pt)
    samp = _sc_gather_call(tbl, fps_idx.reshape(-1))
    idx_t = _knn_call(samp, pt)
    rows_k = _sc_gather_call(tbl, idx_t.reshape(-1))
    rk4 = rows_k.reshape(_B, _K, _S, _W)
    a, qx, sf, qf = _stats_call(rk4, samp)
    return _main_call(rk4, samp, a, qx, sf, qf)


# speedup vs baseline: 11.2304x; 1.0247x over previous
import functools
import math

import jax
import jax.numpy as jnp
import numpy as np
from jax import lax
from jax.experimental import pallas as pl
from jax.experimental.pallas import tpu as pltpu
from jax.experimental.pallas import tpu_sc as plsc

_B = 8
_N = 4096
_S = 1024
_K = 32
_D = 128
_W = 256
_XO = 128
_OUT_DIM = 256
_IN_DIM = 3
_SIGMA = 0.26
_BASELINE = 0.1
_SCALING = 10.0
_EPS = 1e-06
_PAD = 16

_fd = math.ceil(_OUT_DIM / _IN_DIM)
_FEAT_NUM = _fd * _IN_DIM
_OUT_IDX_NP = np.linspace(0, _FEAT_NUM - 1, _OUT_DIM).astype(np.int32)
_FEAT_VAL_NP = np.linspace(-1.0, 1.0, _fd + 2)[1:-1].astype(np.float32)
_SELID_NP = (_OUT_IDX_NP // _fd).astype(np.int32).reshape(1, _OUT_DIM)
_FVSEL_NP = _FEAT_VAL_NP[_OUT_IDX_NP % _fd].astype(np.float32).reshape(1, _OUT_DIM)


def _fps_body(xs_ref, ys_ref, zs_ref, idx_ref):
    xs = xs_ref[...]
    ys = ys_ref[...]
    zs = zs_ref[...]
    lane = lax.broadcasted_iota(jnp.int32, (_B, _N), 1)
    lane_s = lax.broadcasted_iota(jnp.int32, (_B, _S), 1)
    rowoff = lax.broadcasted_iota(jnp.int32, (_B, _S), 0) * _N

    def step(t, carry):
        dist, far, acc = carry
        acc = jnp.where(lane_s == t, jnp.broadcast_to(far, (_B, _S)), acc)
        oh = lane == far
        cx = jnp.sum(jnp.where(oh, xs, 0.0), axis=1, keepdims=True)
        cy = jnp.sum(jnp.where(oh, ys, 0.0), axis=1, keepdims=True)
        cz = jnp.sum(jnp.where(oh, zs, 0.0), axis=1, keepdims=True)
        dx = xs - cx
        dy = ys - cy
        dz = zs - cz
        d = (dx * dx + dy * dy) + dz * dz
        dist = jnp.minimum(dist, d)
        far = jnp.argmax(dist, axis=1).astype(jnp.int32)[:, None]
        return dist, far, acc

    dist0 = xs * 0.0 + 1e10
    far0 = (xs[:, 0:1] * 0.0).astype(jnp.int32)
    acc0 = (xs[:, :_S] * 0.0).astype(jnp.int32)
    _, _, acc = lax.fori_loop(0, _S, step, (dist0, far0, acc0))
    idx_ref[...] = acc + rowoff


def _fps_call(xs, ys, zs):
    return pl.pallas_call(
        _fps_body,
        out_shape=jax.ShapeDtypeStruct((_B, _S), jnp.int32),
    )(xs, ys, zs)


_TQ = 256


def _knn_body(samp_ref, pt_ref, idx_ref):
    b = pl.program_id(0)
    q = samp_ref[:, _XO:_XO + _PAD]
    p = pt_ref[0]
    mm = lax.dot_general(q, p, (((1,), (0,)), ((), ())),
                         preferred_element_type=jnp.float32)
    qx = q[:, 0:1]
    qy = q[:, 1:2]
    qz = q[:, 2:3]
    qn = (qx * qx + qy * qy) + qz * qz
    px = p[0:1, :]
    py = p[1:2, :]
    pz = p[2:3, :]
    pn = (px * px + py * py) + pz * pz
    d = (-2.0 * mm + qn) + pn

    lane = lax.broadcasted_iota(jnp.int32, (_TQ, _N), 1)
    lane_k = lax.broadcasted_iota(jnp.int32, (_TQ, _K), 1)
    boff = b * _N
    acc = jnp.zeros((_TQ, _K), jnp.int32)
    for k in range(_K):
        il = jnp.argmin(d, axis=1).astype(jnp.int32)[:, None]
        acc = jnp.where(lane_k == k, jnp.broadcast_to(il + boff, (_TQ, _K)), acc)
        d = jnp.where(lane == il, jnp.inf, d)
    idx_ref[0] = jnp.transpose(acc)


def _knn_call(samp, pt):
    grid = (_B, _S // _TQ)
    return pl.pallas_call(
        _knn_body,
        grid=grid,
        in_specs=[
            pl.BlockSpec((_TQ, _W), lambda b, st: (b * (_S // _TQ) + st, 0)),
            pl.BlockSpec((1, _PAD, _N), lambda b, st: (b, 0, 0)),
        ],
        out_specs=pl.BlockSpec((1, _K, _TQ), lambda b, st: (b, 0, st)),
        out_shape=jax.ShapeDtypeStruct((_B, _K, _S), jnp.int32),
    )(samp, pt)


_NC = 2
_NS = 16
_NW = _NC * _NS


def _sc_gather_call(tbl, idx_flat):
    n = idx_flat.shape[0]
    per_w = n // _NW
    chunks = per_w // 128
    mesh = plsc.VectorSubcoreMesh(core_axis_name="c", subcore_axis_name="s")

    @functools.partial(
        pl.kernel,
        out_type=jax.ShapeDtypeStruct((n, _W), jnp.float32),
        mesh=mesh,
        scratch_types=[
            pltpu.VMEM((128,), jnp.int32),
            pltpu.VMEM((128,), jnp.int32),
            pltpu.VMEM((128, _W), jnp.float32),
            pltpu.VMEM((128, _W), jnp.float32),
            pltpu.SemaphoreType.DMA,
            pltpu.SemaphoreType.DMA,
            pltpu.SemaphoreType.DMA,
            pltpu.SemaphoreType.DMA,
        ],
    )
    def k(tbl_hbm, idx_hbm, out_hbm, idx_a, idx_b, rows_a, rows_b,
          sem_a, sem_b, osem_a, osem_b):
        wid = lax.axis_index("s") * _NC + lax.axis_index("c")
        idx_v = (idx_a, idx_b)
        rows_v = (rows_a, rows_b)
        sems = (sem_a, sem_b)
        osems = (osem_a, osem_b)

        def pair(c2, _):
            gathers = []
            for bi in range(2):
                base = wid * per_w + (c2 + bi) * 128
                pltpu.sync_copy(idx_hbm.at[pl.ds(base, 128)], idx_v[bi])
                gathers.append(
                    pltpu.async_copy(tbl_hbm.at[idx_v[bi]], rows_v[bi], sems[bi]))
            outs = []
            for bi in range(2):
                base = wid * per_w + (c2 + bi) * 128
                gathers[bi].wait()
                outs.append(
                    pltpu.async_copy(rows_v[bi], out_hbm.at[pl.ds(base, 128)],
                                     osems[bi]))
            for bi in range(2):
                outs[bi].wait()
            return 0

        lax.fori_loop(0, chunks // 2, lambda i, s: pair(i * 2, s), 0)

    return k(tbl, idx_flat)


_TS = 256


def _stats_body(rk_ref, rs_ref, a_ref, qx_ref, sf_ref, qf_ref):
    st = pl.program_id(1)
    rk = rk_ref[0]
    rs = rs_ref[...]
    xk = rk[:, :, _XO:_XO + _PAD]
    xs = rs[:, _XO:_XO + _PAD]
    fk = rk[:, :, :_D]
    fs = rs[:, :_D]

    xd = xk - xs[None]
    a_part = jnp.sum(xd, axis=1)
    q_part = jnp.sum(xd * xd, axis=1)
    fdiff = fk - fs[None]
    sf_part = jnp.sum(jnp.sum(fdiff, axis=2), axis=1)
    qf_part = jnp.sum(jnp.sum(fdiff * fdiff, axis=2), axis=1)

    @pl.when(st == 0)
    def _():
        a_ref[...] = jnp.zeros_like(a_ref)
        qx_ref[...] = jnp.zeros_like(qx_ref)
        sf_ref[...] = jnp.zeros_like(sf_ref)
        qf_ref[...] = jnp.zeros_like(qf_ref)

    a_ref[...] += a_part[None]
    qx_ref[...] += q_part[None]
    sf_ref[...] += sf_part.reshape(1, 1, _K)
    qf_ref[...] += qf_part.reshape(1, 1, _K)


def _stats_call(rk4, samp):
    grid = (_B, _S // _TS)
    return pl.pallas_call(
        _stats_body,
        grid=grid,
        in_specs=[
            pl.BlockSpec((1, _K, _TS, _W), lambda b, st: (b, 0, st, 0)),
            pl.BlockSpec((_TS, _W), lambda b, st: (b * (_S // _TS) + st, 0)),
        ],
        out_specs=[
            pl.BlockSpec((1, _K, _PAD), lambda b, st: (b, 0, 0)),
            pl.BlockSpec((1, _K, _PAD), lambda b, st: (b, 0, 0)),
            pl.BlockSpec((1, 1, _K), lambda b, st: (b, 0, 0)),
            pl.BlockSpec((1, 1, _K), lambda b, st: (b, 0, 0)),
        ],
        out_shape=[
            jax.ShapeDtypeStruct((_B, _K, _PAD), jnp.float32),
            jax.ShapeDtypeStruct((_B, _K, _PAD), jnp.float32),
            jax.ShapeDtypeStruct((_B, 1, _K), jnp.float32),
            jax.ShapeDtypeStruct((_B, 1, _K), jnp.float32),
        ],
    )(rk4, samp)


_TM = 128


def _erf_approx(x):
    ax = jnp.abs(x)
    t = 1.0 / (1.0 + 0.3275911 * ax)
    poly = t * (0.254829592 + t * (-0.284496736 + t * (1.421413741
        + t * (-1.453152027 + t * 1.061405429))))
    y = 1.0 - poly * jnp.exp(-ax * ax)
    return jnp.sign(x) * y


def _gelu_exact(x):
    return 0.5 * x * (1.0 + _erf_approx(x * np.float32(1.0 / math.sqrt(2.0))))


def _main_body(rk_ref, rs_ref, a_ref, qx_ref, sf_ref, qf_ref,
               fv_ref, sid_ref, o_ref):
    a = a_ref[...]
    qx = qx_ref[...]
    nx = np.float32(_B * _S * _IN_DIM)
    sum_a = jnp.sum(jnp.sum(a, axis=0), axis=1, keepdims=True)
    sum_q = jnp.sum(jnp.sum(qx, axis=0), axis=1, keepdims=True)
    var_x = (sum_q - sum_a * sum_a / nx) / (nx - 1.0)
    std_x = jnp.maximum(jnp.sqrt(jnp.maximum(var_x, 0.0)), 1e-05)
    inv_x = 1.0 / std_x

    sk = np.float32(_S * _K)
    an = a * inv_x[None]
    qn = qx * (inv_x * inv_x)[None]
    sum_bd = jnp.sum(an, axis=1)
    ssq_bd = jnp.sum(qn, axis=1)
    var_bd = (ssq_bd - sum_bd * sum_bd / sk) / (sk - 1.0)
    gs = jnp.sum(jnp.sqrt(jnp.maximum(var_bd, 0.0))) / np.float32(_B * _IN_DIM)
    sigma = _SIGMA * (1.0 + gs)
    r = 1.0 / (sigma + _EPS)
    blend = 1.0 / (1.0 + jnp.exp(-(gs - _BASELINE) * _SCALING))
    one_m_blend = 1.0 - blend

    nf = np.float32(_B * _S * _D)
    sum_f = jnp.sum(sf_ref[...], axis=0)
    sum_qf = jnp.sum(qf_ref[...], axis=0)
    var_f = (sum_qf - sum_f * sum_f / nf) / (nf - 1.0)
    std_f = jnp.maximum(jnp.sqrt(jnp.maximum(var_f, 0.0)), 1e-05)
    inv_f = 1.0 / std_f

    fv = fv_ref[...]
    sid = sid_ref[...]
    rs = rs_ref[...]
    fs = rs[:, :_D]
    xs = rs[:, _XO:_XO + _PAD]

    neg_inf = np.float32(-np.inf)
    cfv = one_m_blend * jnp.cos(fv * r)
    sfv = one_m_blend * jnp.sin(fv * r)
    s_acc = [jnp.zeros((_TM, _D), jnp.float32) for _ in range(2)]
    m_acc = [jnp.full((_TM, _D), neg_inf, jnp.float32) for _ in range(2)]
    for k in range(_K):
        row = rk_ref[0, k]
        xn = (row[:, _XO:_XO + _PAD] - xs) * inv_x[k:k + 1, 0:1]
        rx = xn * r
        crx = jnp.cos(rx)
        srx = jnp.sin(rx)
        for half in range(2):
            fvh = fv[:, half * _D:(half + 1) * _D]
            sidh = sid[:, half * _D:(half + 1) * _D]
            xsel = jnp.where(sidh == 0, xn[:, 0:1],
                             jnp.where(sidh == 1, xn[:, 1:2], xn[:, 2:3]))
            csel = jnp.where(sidh == 0, crx[:, 0:1],
                             jnp.where(sidh == 1, crx[:, 1:2], crx[:, 2:3]))
            ssel = jnp.where(sidh == 0, srx[:, 0:1],
                             jnp.where(sidh == 1, srx[:, 1:2], srx[:, 2:3]))
            t = (xsel - fvh) * r
            pe = (blend * jnp.exp(-0.5 * (t * t))
                  + (csel * cfv[:, half * _D:(half + 1) * _D]
                     + ssel * sfv[:, half * _D:(half + 1) * _D]))
            if half == 0:
                fc = (row[:, :_D] - fs) * inv_f[0:1, k:k + 1]
            else:
                fc = fs
            w = (fc + pe) * pe
            s_acc[half] = s_acc[half] + w
            m_acc[half] = jnp.maximum(m_acc[half], w)
    for half in range(2):
        agg = s_acc[half] * np.float32(1.0 / _K) + m_acc[half]
        o_ref[0, :, half * _D:(half + 1) * _D] = _gelu_exact(agg)


def _main_call(rk4, samp, a, qx, sf, qf):
    grid = (_B, _S // _TM)
    fv = jnp.asarray(_FVSEL_NP)
    sid = jnp.asarray(_SELID_NP)
    return pl.pallas_call(
        _main_body,
        grid=grid,
        in_specs=[
            pl.BlockSpec((1, _K, _TM, _W), lambda b, st: (b, 0, st, 0)),
            pl.BlockSpec((_TM, _W), lambda b, st: (b * (_S // _TM) + st, 0)),
            pl.BlockSpec((_B, _K, _PAD), lambda b, st: (0, 0, 0)),
            pl.BlockSpec((_B, _K, _PAD), lambda b, st: (0, 0, 0)),
            pl.BlockSpec((_B, 1, _K), lambda b, st: (0, 0, 0)),
            pl.BlockSpec((_B, 1, _K), lambda b, st: (0, 0, 0)),
            pl.BlockSpec((1, _OUT_DIM), lambda b, st: (0, 0)),
            pl.BlockSpec((1, _OUT_DIM), lambda b, st: (0, 0)),
        ],
        out_specs=pl.BlockSpec((1, _TM, _OUT_DIM), lambda b, st: (b, st, 0)),
        out_shape=jax.ShapeDtypeStruct((_B, _S, _OUT_DIM), jnp.float32),
    )(rk4, samp, a, qx, sf, qf, fv, sid)


def kernel(xyz, feat):
    xs = xyz[:, :, 0]
    ys = xyz[:, :, 1]
    zs = xyz[:, :, 2]
    xyzp = jnp.pad(xyz, ((0, 0), (0, 0), (0, _PAD - _IN_DIM)))
    pt = jnp.transpose(xyzp, (0, 2, 1))
    tbl = jnp.concatenate(
        [feat, jnp.pad(xyz, ((0, 0), (0, 0), (0, _W - _D - _IN_DIM)))],
        axis=-1).reshape(_B * _N, _W)

    fps_idx = _fps_call(xs, ys, zs)
    samp = _sc_gather_call(tbl, fps_idx.reshape(-1))
    idx_t = _knn_call(samp, pt)
    samp = _sc_gather_call(tbl, fps_idx.reshape(-1))
    idx_t = _knn_call(samp, pt)
    rows_k = _sc_gather_call(tbl, idx_t.reshape(-1))
    rk4 = rows_k.reshape(_B, _K, _S, _W)
    a, qx, sf, qf = _stats_call(rk4, samp)
    return _main_call(rk4, samp, a, qx, sf, qf)


# speedup vs baseline: 11.8683x; 1.0568x over previous
import functools
import math

import jax
import jax.numpy as jnp
import numpy as np
from jax import lax
from jax.experimental import pallas as pl
from jax.experimental.pallas import tpu as pltpu
from jax.experimental.pallas import tpu_sc as plsc

_B = 8
_N = 4096
_S = 1024
_K = 32
_D = 128
_W = 256
_XO = 128
_OUT_DIM = 256
_IN_DIM = 3
_SIGMA = 0.26
_BASELINE = 0.1
_SCALING = 10.0
_EPS = 1e-06
_PAD = 16

_fd = math.ceil(_OUT_DIM / _IN_DIM)
_FEAT_NUM = _fd * _IN_DIM
_OUT_IDX_NP = np.linspace(0, _FEAT_NUM - 1, _OUT_DIM).astype(np.int32)
_FEAT_VAL_NP = np.linspace(-1.0, 1.0, _fd + 2)[1:-1].astype(np.float32)
_SELID_NP = (_OUT_IDX_NP // _fd).astype(np.int32).reshape(1, _OUT_DIM)
_FVSEL_NP = _FEAT_VAL_NP[_OUT_IDX_NP % _fd].astype(np.float32).reshape(1, _OUT_DIM)
_SEL01_NP = np.zeros((_PAD, _OUT_DIM), np.float32)
_SEL01_NP[_SELID_NP[0], np.arange(_OUT_DIM)] = 1.0


def _fps_body(xs_ref, ys_ref, zs_ref, idx_ref):
    xs = xs_ref[...]
    ys = ys_ref[...]
    zs = zs_ref[...]
    lane = lax.broadcasted_iota(jnp.int32, (_B, _N), 1)
    lane_s = lax.broadcasted_iota(jnp.int32, (_B, _S), 1)
    rowoff = lax.broadcasted_iota(jnp.int32, (_B, _S), 0) * _N

    def step(t, carry):
        dist, far, acc = carry
        acc = jnp.where(lane_s == t, jnp.broadcast_to(far, (_B, _S)), acc)
        oh = lane == far
        cx = jnp.sum(jnp.where(oh, xs, 0.0), axis=1, keepdims=True)
        cy = jnp.sum(jnp.where(oh, ys, 0.0), axis=1, keepdims=True)
        cz = jnp.sum(jnp.where(oh, zs, 0.0), axis=1, keepdims=True)
        dx = xs - cx
        dy = ys - cy
        dz = zs - cz
        d = (dx * dx + dy * dy) + dz * dz
        dist = jnp.minimum(dist, d)
        far = jnp.argmax(dist, axis=1).astype(jnp.int32)[:, None]
        return dist, far, acc

    dist0 = xs * 0.0 + 1e10
    far0 = (xs[:, 0:1] * 0.0).astype(jnp.int32)
    acc0 = (xs[:, :_S] * 0.0).astype(jnp.int32)
    _, _, acc = lax.fori_loop(0, _S, step, (dist0, far0, acc0))
    idx_ref[...] = acc + rowoff


def _fps_call(xs, ys, zs):
    return pl.pallas_call(
        _fps_body,
        out_shape=jax.ShapeDtypeStruct((_B, _S), jnp.int32),
    )(xs, ys, zs)


_TQ = 256


def _knn_body(samp_ref, pt_ref, idx_ref):
    b = pl.program_id(0)
    q = samp_ref[:, _XO:_XO + _PAD]
    p = pt_ref[0]
    mm = lax.dot_general(q, p, (((1,), (0,)), ((), ())),
                         preferred_element_type=jnp.float32)
    qx = q[:, 0:1]
    qy = q[:, 1:2]
    qz = q[:, 2:3]
    qn = (qx * qx + qy * qy) + qz * qz
    px = p[0:1, :]
    py = p[1:2, :]
    pz = p[2:3, :]
    pn = (px * px + py * py) + pz * pz
    d = (-2.0 * mm + qn) + pn

    lane = lax.broadcasted_iota(jnp.int32, (_TQ, _N), 1)
    lane_k = lax.broadcasted_iota(jnp.int32, (_TQ, _K), 1)
    boff = b * _N
    acc = jnp.zeros((_TQ, _K), jnp.int32)
    for k in range(_K):
        il = jnp.argmin(d, axis=1).astype(jnp.int32)[:, None]
        acc = jnp.where(lane_k == k, jnp.broadcast_to(il + boff, (_TQ, _K)), acc)
        d = jnp.where(lane == il, jnp.inf, d)
    idx_ref[0] = jnp.transpose(acc)


def _knn_call(samp, pt):
    grid = (_B, _S // _TQ)
    return pl.pallas_call(
        _knn_body,
        grid=grid,
        in_specs=[
            pl.BlockSpec((_TQ, _W), lambda b, st: (b * (_S // _TQ) + st, 0)),
            pl.BlockSpec((1, _PAD, _N), lambda b, st: (b, 0, 0)),
        ],
        out_specs=pl.BlockSpec((1, _K, _TQ), lambda b, st: (b, 0, st)),
        out_shape=jax.ShapeDtypeStruct((_B, _K, _S), jnp.int32),
    )(samp, pt)


_NC = 2
_NS = 16
_NW = _NC * _NS


def _sc_gather_call(tbl, idx_flat):
    n = idx_flat.shape[0]
    per_w = n // _NW
    chunks = per_w // 128
    mesh = plsc.VectorSubcoreMesh(core_axis_name="c", subcore_axis_name="s")

    @functools.partial(
        pl.kernel,
        out_type=jax.ShapeDtypeStruct((n, _W), jnp.float32),
        mesh=mesh,
        scratch_types=[
            pltpu.VMEM((128,), jnp.int32),
            pltpu.VMEM((128,), jnp.int32),
            pltpu.VMEM((128, _W), jnp.float32),
            pltpu.VMEM((128, _W), jnp.float32),
            pltpu.SemaphoreType.DMA,
            pltpu.SemaphoreType.DMA,
            pltpu.SemaphoreType.DMA,
            pltpu.SemaphoreType.DMA,
        ],
    )
    def k(tbl_hbm, idx_hbm, out_hbm, idx_a, idx_b, rows_a, rows_b,
          sem_a, sem_b, osem_a, osem_b):
        wid = lax.axis_index("s") * _NC + lax.axis_index("c")
        idx_v = (idx_a, idx_b)
        rows_v = (rows_a, rows_b)
        sems = (sem_a, sem_b)
        osems = (osem_a, osem_b)

        def pair(c2, _):
            gathers = []
            for bi in range(2):
                base = wid * per_w + (c2 + bi) * 128
                pltpu.sync_copy(idx_hbm.at[pl.ds(base, 128)], idx_v[bi])
                gathers.append(
                    pltpu.async_copy(tbl_hbm.at[idx_v[bi]], rows_v[bi], sems[bi]))
            outs = []
            for bi in range(2):
                base = wid * per_w + (c2 + bi) * 128
                gathers[bi].wait()
                outs.append(
                    pltpu.async_copy(rows_v[bi], out_hbm.at[pl.ds(base, 128)],
                                     osems[bi]))
            for bi in range(2):
                outs[bi].wait()
            return 0

        lax.fori_loop(0, chunks // 2, lambda i, s: pair(i * 2, s), 0)

    return k(tbl, idx_flat)


_TS = 256


def _stats_body(rk_ref, rs_ref, a_ref, qx_ref, sf_ref, qf_ref):
    st = pl.program_id(1)
    rk = rk_ref[0]
    rs = rs_ref[...]
    xk = rk[:, :, _XO:_XO + _PAD]
    xs = rs[:, _XO:_XO + _PAD]
    fk = rk[:, :, :_D]
    fs = rs[:, :_D]

    xd = xk - xs[None]
    a_part = jnp.sum(xd, axis=1)
    q_part = jnp.sum(xd * xd, axis=1)
    fdiff = fk - fs[None]
    sf_part = jnp.sum(jnp.sum(fdiff, axis=2), axis=1)
    qf_part = jnp.sum(jnp.sum(fdiff * fdiff, axis=2), axis=1)

    @pl.when(st == 0)
    def _():
        a_ref[...] = jnp.zeros_like(a_ref)
        qx_ref[...] = jnp.zeros_like(qx_ref)
        sf_ref[...] = jnp.zeros_like(sf_ref)
        qf_ref[...] = jnp.zeros_like(qf_ref)

    a_ref[...] += a_part[None]
    qx_ref[...] += q_part[None]
    sf_ref[...] += sf_part.reshape(1, 1, _K)
    qf_ref[...] += qf_part.reshape(1, 1, _K)


def _stats_call(rk4, samp):
    grid = (_B, _S // _TS)
    return pl.pallas_call(
        _stats_body,
        grid=grid,
        in_specs=[
            pl.BlockSpec((1, _K, _TS, _W), lambda b, st: (b, 0, st, 0)),
            pl.BlockSpec((_TS, _W), lambda b, st: (b * (_S // _TS) + st, 0)),
        ],
        out_specs=[
            pl.BlockSpec((1, _K, _PAD), lambda b, st: (b, 0, 0)),
            pl.BlockSpec((1, _K, _PAD), lambda b, st: (b, 0, 0)),
            pl.BlockSpec((1, 1, _K), lambda b, st: (b, 0, 0)),
            pl.BlockSpec((1, 1, _K), lambda b, st: (b, 0, 0)),
        ],
        out_shape=[
            jax.ShapeDtypeStruct((_B, _K, _PAD), jnp.float32),
            jax.ShapeDtypeStruct((_B, _K, _PAD), jnp.float32),
            jax.ShapeDtypeStruct((_B, 1, _K), jnp.float32),
            jax.ShapeDtypeStruct((_B, 1, _K), jnp.float32),
        ],
    )(rk4, samp)


_TM = 128


def _erf_approx(x):
    ax = jnp.abs(x)
    t = 1.0 / (1.0 + 0.3275911 * ax)
    poly = t * (0.254829592 + t * (-0.284496736 + t * (1.421413741
        + t * (-1.453152027 + t * 1.061405429))))
    y = 1.0 - poly * jnp.exp(-ax * ax)
    return jnp.sign(x) * y


def _gelu_exact(x):
    return 0.5 * x * (1.0 + _erf_approx(x * np.float32(1.0 / math.sqrt(2.0))))


def _main_body(rk_ref, rs_ref, a_ref, qx_ref, sf_ref, qf_ref,
               fv_ref, sel_ref, o_ref):
    a = a_ref[...]
    qx = qx_ref[...]
    nx = np.float32(_B * _S * _IN_DIM)
    sum_a = jnp.sum(jnp.sum(a, axis=0), axis=1, keepdims=True)
    sum_q = jnp.sum(jnp.sum(qx, axis=0), axis=1, keepdims=True)
    var_x = (sum_q - sum_a * sum_a / nx) / (nx - 1.0)
    std_x = jnp.maximum(jnp.sqrt(jnp.maximum(var_x, 0.0)), 1e-05)
    inv_x = 1.0 / std_x

    sk = np.float32(_S * _K)
    an = a * inv_x[None]
    qn = qx * (inv_x * inv_x)[None]
    sum_bd = jnp.sum(an, axis=1)
    ssq_bd = jnp.sum(qn, axis=1)
    var_bd = (ssq_bd - sum_bd * sum_bd / sk) / (sk - 1.0)
    gs = jnp.sum(jnp.sqrt(jnp.maximum(var_bd, 0.0))) / np.float32(_B * _IN_DIM)
    sigma = _SIGMA * (1.0 + gs)
    r = 1.0 / (sigma + _EPS)
    blend = 1.0 / (1.0 + jnp.exp(-(gs - _BASELINE) * _SCALING))
    one_m_blend = 1.0 - blend

    nf = np.float32(_B * _S * _D)
    sum_f = jnp.sum(sf_ref[...], axis=0)
    sum_qf = jnp.sum(qf_ref[...], axis=0)
    var_f = (sum_qf - sum_f * sum_f / nf) / (nf - 1.0)
    std_f = jnp.maximum(jnp.sqrt(jnp.maximum(var_f, 0.0)), 1e-05)
    inv_f = 1.0 / std_f

    fv = fv_ref[...]
    rs = rs_ref[...]
    fs = rs[:, :_D]
    xs = rs[:, _XO:_XO + _PAD]

    neg_inf = np.float32(-np.inf)
    sel01 = sel_ref[...]
    rfv = fv * r
    mc = sel01 * (one_m_blend * jnp.cos(rfv))
    ms = sel01 * (one_m_blend * jnp.sin(rfv))
    dn = (((1,), (0,)), ((), ()))
    s_acc = [jnp.zeros((_TM, _D), jnp.float32) for _ in range(2)]
    m_acc = [jnp.full((_TM, _D), neg_inf, jnp.float32) for _ in range(2)]
    for k in range(_K):
        row = rk_ref[0, k]
        xn = (row[:, _XO:_XO + _PAD] - xs) * inv_x[k:k + 1, 0:1]
        rx = xn * r
        crx = jnp.cos(rx)
        srx = jnp.sin(rx)
        t = lax.dot_general(rx, sel01, dn,
                            preferred_element_type=jnp.float32) - rfv
        cospart = (lax.dot_general(crx, mc, dn, preferred_element_type=jnp.float32)
                   + lax.dot_general(srx, ms, dn, preferred_element_type=jnp.float32))
        pe_full = blend * jnp.exp(-0.5 * (t * t)) + cospart
        for half in range(2):
            pe = pe_full[:, half * _D:(half + 1) * _D]
            if half == 0:
                fc = (row[:, :_D] - fs) * inv_f[0:1, k:k + 1]
            else:
                fc = fs
            w = (fc + pe) * pe
            s_acc[half] = s_acc[half] + w
            m_acc[half] = jnp.maximum(m_acc[half], w)
    for half in range(2):
        agg = s_acc[half] * np.float32(1.0 / _K) + m_acc[half]
        o_ref[0, :, half * _D:(half + 1) * _D] = _gelu_exact(agg)


def _main_call(rk4, samp, a, qx, sf, qf):
    grid = (_B, _S // _TM)
    fv = jnp.asarray(_FVSEL_NP)
    sel01 = jnp.asarray(_SEL01_NP)
    return pl.pallas_call(
        _main_body,
        grid=grid,
        in_specs=[
            pl.BlockSpec((1, _K, _TM, _W), lambda b, st: (b, 0, st, 0)),
            pl.BlockSpec((_TM, _W), lambda b, st: (b * (_S // _TM) + st, 0)),
            pl.BlockSpec((_B, _K, _PAD), lambda b, st: (0, 0, 0)),
            pl.BlockSpec((_B, _K, _PAD), lambda b, st: (0, 0, 0)),
            pl.BlockSpec((_B, 1, _K), lambda b, st: (0, 0, 0)),
            pl.BlockSpec((_B, 1, _K), lambda b, st: (0, 0, 0)),
            pl.BlockSpec((1, _OUT_DIM), lambda b, st: (0, 0)),
            pl.BlockSpec((_PAD, _OUT_DIM), lambda b, st: (0, 0)),
        ],
        out_specs=pl.BlockSpec((1, _TM, _OUT_DIM), lambda b, st: (b, st, 0)),
        out_shape=jax.ShapeDtypeStruct((_B, _S, _OUT_DIM), jnp.float32),
    )(rk4, samp, a, qx, sf, qf, fv, sel01)


def kernel(xyz, feat):
    xs = xyz[:, :, 0]
    ys = xyz[:, :, 1]
    zs = xyz[:, :, 2]
    xyzp = jnp.pad(xyz, ((0, 0), (0, 0), (0, _PAD - _IN_DIM)))
    pt = jnp.transpose(xyzp, (0, 2, 1))
    tbl = jnp.concatenate(
        [feat, jnp.pad(xyz, ((0, 0), (0, 0), (0, _W - _D - _IN_DIM)))],
        axis=-1).reshape(_B * _N, _W)

    fps_idx = _fps_call(xs, ys, zs)
    samp = _sc_gather_call(tbl, fps_idx.reshape(-1))
    idx_t = _knn_call(samp, pt)
    samp = _sc_gather_call(tbl, fps_idx.reshape(-1))
    idx_t = _knn_call(samp, pt)
    rows_k = _sc_gather_call(tbl, idx_t.reshape(-1))
    rk4 = rows_k.reshape(_B, _K, _S, _W)
    a, qx, sf, qf = _stats_call(rk4, samp)
    return _main_call(rk4, samp, a, qx, sf, qf)


# speedup vs baseline: 11.9970x; 1.0108x over previous
import functools
import math

import jax
import jax.numpy as jnp
import numpy as np
from jax import lax
from jax.experimental import pallas as pl
from jax.experimental.pallas import tpu as pltpu
from jax.experimental.pallas import tpu_sc as plsc

_B = 8
_N = 4096
_S = 1024
_K = 32
_D = 128
_W = 256
_XO = 128
_OUT_DIM = 256
_IN_DIM = 3
_SIGMA = 0.26
_BASELINE = 0.1
_SCALING = 10.0
_EPS = 1e-06
_PAD = 16

_fd = math.ceil(_OUT_DIM / _IN_DIM)
_FEAT_NUM = _fd * _IN_DIM
_OUT_IDX_NP = np.linspace(0, _FEAT_NUM - 1, _OUT_DIM).astype(np.int32)
_FEAT_VAL_NP = np.linspace(-1.0, 1.0, _fd + 2)[1:-1].astype(np.float32)
_SELID_NP = (_OUT_IDX_NP // _fd).astype(np.int32).reshape(1, _OUT_DIM)
_FVSEL_NP = _FEAT_VAL_NP[_OUT_IDX_NP % _fd].astype(np.float32).reshape(1, _OUT_DIM)
_SEL01_NP = np.zeros((_PAD, _OUT_DIM), np.float32)
_SEL01_NP[_SELID_NP[0], np.arange(_OUT_DIM)] = 1.0


def _fps_body(xs_ref, ys_ref, zs_ref, idx_ref):
    xs = xs_ref[...]
    ys = ys_ref[...]
    zs = zs_ref[...]
    lane = lax.broadcasted_iota(jnp.int32, (_B, _N), 1)
    lane_s = lax.broadcasted_iota(jnp.int32, (_B, _S), 1)
    rowoff = lax.broadcasted_iota(jnp.int32, (_B, _S), 0) * _N

    def step(t, carry):
        dist, far, acc = carry
        acc = jnp.where(lane_s == t, jnp.broadcast_to(far, (_B, _S)), acc)
        oh = lane == far
        cx = jnp.sum(jnp.where(oh, xs, 0.0), axis=1, keepdims=True)
        cy = jnp.sum(jnp.where(oh, ys, 0.0), axis=1, keepdims=True)
        cz = jnp.sum(jnp.where(oh, zs, 0.0), axis=1, keepdims=True)
        dx = xs - cx
        dy = ys - cy
        dz = zs - cz
        d = (dx * dx + dy * dy) + dz * dz
        dist = jnp.minimum(dist, d)
        far = jnp.argmax(dist, axis=1).astype(jnp.int32)[:, None]
        return dist, far, acc

    dist0 = xs * 0.0 + 1e10
    far0 = (xs[:, 0:1] * 0.0).astype(jnp.int32)
    acc0 = (xs[:, :_S] * 0.0).astype(jnp.int32)
    _, _, acc = lax.fori_loop(0, _S, step, (dist0, far0, acc0))
    idx_ref[...] = acc + rowoff


def _fps_call(xs, ys, zs):
    return pl.pallas_call(
        _fps_body,
        out_shape=jax.ShapeDtypeStruct((_B, _S), jnp.int32),
    )(xs, ys, zs)


_TQ = 256


def _knn_body(samp_ref, pt_ref, idx_ref):
    b = pl.program_id(0)
    q = samp_ref[:, _XO:_XO + _PAD]
    p = pt_ref[0]
    mm = lax.dot_general(q, p, (((1,), (0,)), ((), ())),
                         preferred_element_type=jnp.float32)
    qx = q[:, 0:1]
    qy = q[:, 1:2]
    qz = q[:, 2:3]
    qn = (qx * qx + qy * qy) + qz * qz
    px = p[0:1, :]
    py = p[1:2, :]
    pz = p[2:3, :]
    pn = (px * px + py * py) + pz * pz
    d = (-2.0 * mm + qn) + pn

    lane = lax.broadcasted_iota(jnp.int32, (_TQ, _N), 1)
    lane_k = lax.broadcasted_iota(jnp.int32, (_TQ, _K), 1)
    boff = b * _N
    acc = jnp.zeros((_TQ, _K), jnp.int32)
    for k in range(_K):
        il = jnp.argmin(d, axis=1).astype(jnp.int32)[:, None]
        acc = jnp.where(lane_k == k, jnp.broadcast_to(il + boff, (_TQ, _K)), acc)
        d = jnp.where(lane == il, jnp.inf, d)
    idx_ref[0] = jnp.transpose(acc)


def _knn_call(samp, pt):
    grid = (_B, _S // _TQ)
    return pl.pallas_call(
        _knn_body,
        grid=grid,
        in_specs=[
            pl.BlockSpec((_TQ, _W), lambda b, st: (b * (_S // _TQ) + st, 0)),
            pl.BlockSpec((1, _PAD, _N), lambda b, st: (b, 0, 0)),
        ],
        out_specs=pl.BlockSpec((1, _K, _TQ), lambda b, st: (b, 0, st)),
        out_shape=jax.ShapeDtypeStruct((_B, _K, _S), jnp.int32),
    )(samp, pt)


_NC = 2
_NS = 16
_NW = _NC * _NS


def _sc_gather_call(tbl, idx_flat):
    n = idx_flat.shape[0]
    per_w = n // _NW
    chunks = per_w // 128
    mesh = plsc.VectorSubcoreMesh(core_axis_name="c", subcore_axis_name="s")

    @functools.partial(
        pl.kernel,
        out_type=jax.ShapeDtypeStruct((n, _W), jnp.float32),
        mesh=mesh,
        scratch_types=[
            pltpu.VMEM((128,), jnp.int32),
            pltpu.VMEM((128,), jnp.int32),
            pltpu.VMEM((128, _W), jnp.float32),
            pltpu.VMEM((128, _W), jnp.float32),
            pltpu.SemaphoreType.DMA,
            pltpu.SemaphoreType.DMA,
            pltpu.SemaphoreType.DMA,
            pltpu.SemaphoreType.DMA,
        ],
    )
    def k(tbl_hbm, idx_hbm, out_hbm, idx_a, idx_b, rows_a, rows_b,
          sem_a, sem_b, osem_a, osem_b):
        wid = lax.axis_index("s") * _NC + lax.axis_index("c")
        idx_v = (idx_a, idx_b)
        rows_v = (rows_a, rows_b)
        sems = (sem_a, sem_b)
        osems = (osem_a, osem_b)

        def pair(c2, _):
            gathers = []
            for bi in range(2):
                base = wid * per_w + (c2 + bi) * 128
                pltpu.sync_copy(idx_hbm.at[pl.ds(base, 128)], idx_v[bi])
                gathers.append(
                    pltpu.async_copy(tbl_hbm.at[idx_v[bi]], rows_v[bi], sems[bi]))
            outs = []
            for bi in range(2):
                base = wid * per_w + (c2 + bi) * 128
                gathers[bi].wait()
                outs.append(
                    pltpu.async_copy(rows_v[bi], out_hbm.at[pl.ds(base, 128)],
                                     osems[bi]))
            for bi in range(2):
                outs[bi].wait()
            return 0

        lax.fori_loop(0, chunks // 2, lambda i, s: pair(i * 2, s), 0)

    return k(tbl, idx_flat)


_TS = 256


def _stats_body(rk_ref, rs_ref, a_ref, qx_ref, sf_ref, qf_ref):
    st = pl.program_id(1)
    rk = rk_ref[0]
    rs = rs_ref[...]
    xk = rk[:, :, _XO:_XO + _PAD]
    xs = rs[:, _XO:_XO + _PAD]
    fk = rk[:, :, :_D]
    fs = rs[:, :_D]

    xd = xk - xs[None]
    a_part = jnp.sum(xd, axis=1)
    q_part = jnp.sum(xd * xd, axis=1)
    fdiff = fk - fs[None]
    sf_part = jnp.sum(jnp.sum(fdiff, axis=2), axis=1)
    qf_part = jnp.sum(jnp.sum(fdiff * fdiff, axis=2), axis=1)

    @pl.when(st == 0)
    def _():
        a_ref[...] = jnp.zeros_like(a_ref)
        qx_ref[...] = jnp.zeros_like(qx_ref)
        sf_ref[...] = jnp.zeros_like(sf_ref)
        qf_ref[...] = jnp.zeros_like(qf_ref)

    a_ref[...] += a_part[None]
    qx_ref[...] += q_part[None]
    sf_ref[...] += sf_part.reshape(1, 1, _K)
    qf_ref[...] += qf_part.reshape(1, 1, _K)


def _stats_call(rk4, samp):
    grid = (_B, _S // _TS)
    return pl.pallas_call(
        _stats_body,
        grid=grid,
        in_specs=[
            pl.BlockSpec((1, _K, _TS, _W), lambda b, st: (b, 0, st, 0)),
            pl.BlockSpec((_TS, _W), lambda b, st: (b * (_S // _TS) + st, 0)),
        ],
        out_specs=[
            pl.BlockSpec((1, _K, _PAD), lambda b, st: (b, 0, 0)),
            pl.BlockSpec((1, _K, _PAD), lambda b, st: (b, 0, 0)),
            pl.BlockSpec((1, 1, _K), lambda b, st: (b, 0, 0)),
            pl.BlockSpec((1, 1, _K), lambda b, st: (b, 0, 0)),
        ],
        out_shape=[
            jax.ShapeDtypeStruct((_B, _K, _PAD), jnp.float32),
            jax.ShapeDtypeStruct((_B, _K, _PAD), jnp.float32),
            jax.ShapeDtypeStruct((_B, 1, _K), jnp.float32),
            jax.ShapeDtypeStruct((_B, 1, _K), jnp.float32),
        ],
    )(rk4, samp)


_TM = 256


def _erf_approx(x):
    ax = jnp.abs(x)
    t = 1.0 / (1.0 + 0.3275911 * ax)
    poly = t * (0.254829592 + t * (-0.284496736 + t * (1.421413741
        + t * (-1.453152027 + t * 1.061405429))))
    y = 1.0 - poly * jnp.exp(-ax * ax)
    return jnp.sign(x) * y


def _gelu_exact(x):
    return 0.5 * x * (1.0 + _erf_approx(x * np.float32(1.0 / math.sqrt(2.0))))


def _main_body(rk_ref, rs_ref, a_ref, qx_ref, sf_ref, qf_ref,
               fv_ref, sel_ref, o_ref):
    a = a_ref[...]
    qx = qx_ref[...]
    nx = np.float32(_B * _S * _IN_DIM)
    sum_a = jnp.sum(jnp.sum(a, axis=0), axis=1, keepdims=True)
    sum_q = jnp.sum(jnp.sum(qx, axis=0), axis=1, keepdims=True)
    var_x = (sum_q - sum_a * sum_a / nx) / (nx - 1.0)
    std_x = jnp.maximum(jnp.sqrt(jnp.maximum(var_x, 0.0)), 1e-05)
    inv_x = 1.0 / std_x

    sk = np.float32(_S * _K)
    an = a * inv_x[None]
    qn = qx * (inv_x * inv_x)[None]
    sum_bd = jnp.sum(an, axis=1)
    ssq_bd = jnp.sum(qn, axis=1)
    var_bd = (ssq_bd - sum_bd * sum_bd / sk) / (sk - 1.0)
    gs = jnp.sum(jnp.sqrt(jnp.maximum(var_bd, 0.0))) / np.float32(_B * _IN_DIM)
    sigma = _SIGMA * (1.0 + gs)
    r = 1.0 / (sigma + _EPS)
    blend = 1.0 / (1.0 + jnp.exp(-(gs - _BASELINE) * _SCALING))
    one_m_blend = 1.0 - blend

    nf = np.float32(_B * _S * _D)
    sum_f = jnp.sum(sf_ref[...], axis=0)
    sum_qf = jnp.sum(qf_ref[...], axis=0)
    var_f = (sum_qf - sum_f * sum_f / nf) / (nf - 1.0)
    std_f = jnp.maximum(jnp.sqrt(jnp.maximum(var_f, 0.0)), 1e-05)
    inv_f = 1.0 / std_f

    fv = fv_ref[...]
    rs = rs_ref[...]
    fs = rs[:, :_D]
    xs = rs[:, _XO:_XO + _PAD]

    neg_inf = np.float32(-np.inf)
    sel01 = sel_ref[...]
    rfv = fv * r
    mc = sel01 * (one_m_blend * jnp.cos(rfv))
    ms = sel01 * (one_m_blend * jnp.sin(rfv))
    dn = (((1,), (0,)), ((), ()))
    s_acc = [jnp.zeros((_TM, _D), jnp.float32) for _ in range(2)]
    m_acc = [jnp.full((_TM, _D), neg_inf, jnp.float32) for _ in range(2)]
    for k in range(_K):
        row = rk_ref[0, k]
        xn = (row[:, _XO:_XO + _PAD] - xs) * inv_x[k:k + 1, 0:1]
        rx = xn * r
        crx = jnp.cos(rx)
        srx = jnp.sin(rx)
        t = lax.dot_general(rx, sel01, dn,
                            preferred_element_type=jnp.float32) - rfv
        cospart = (lax.dot_general(crx, mc, dn, preferred_element_type=jnp.float32)
                   + lax.dot_general(srx, ms, dn, preferred_element_type=jnp.float32))
        pe_full = blend * jnp.exp(-0.5 * (t * t)) + cospart
        for half in range(2):
            pe = pe_full[:, half * _D:(half + 1) * _D]
            if half == 0:
                fc = (row[:, :_D] - fs) * inv_f[0:1, k:k + 1]
            else:
                fc = fs
            w = (fc + pe) * pe
            s_acc[half] = s_acc[half] + w
            m_acc[half] = jnp.maximum(m_acc[half], w)
    for half in range(2):
        agg = s_acc[half] * np.float32(1.0 / _K) + m_acc[half]
        o_ref[0, :, half * _D:(half + 1) * _D] = _gelu_exact(agg)


def _main_call(rk4, samp, a, qx, sf, qf):
    grid = (_B, _S // _TM)
    fv = jnp.asarray(_FVSEL_NP)
    sel01 = jnp.asarray(_SEL01_NP)
    return pl.pallas_call(
        _main_body,
        grid=grid,
        in_specs=[
            pl.BlockSpec((1, _K, _TM, _W), lambda b, st: (b, 0, st, 0)),
            pl.BlockSpec((_TM, _W), lambda b, st: (b * (_S // _TM) + st, 0)),
            pl.BlockSpec((_B, _K, _PAD), lambda b, st: (0, 0, 0)),
            pl.BlockSpec((_B, _K, _PAD), lambda b, st: (0, 0, 0)),
            pl.BlockSpec((_B, 1, _K), lambda b, st: (0, 0, 0)),
            pl.BlockSpec((_B, 1, _K), lambda b, st: (0, 0, 0)),
            pl.BlockSpec((1, _OUT_DIM), lambda b, st: (0, 0)),
            pl.BlockSpec((_PAD, _OUT_DIM), lambda b, st: (0, 0)),
        ],
        out_specs=pl.BlockSpec((1, _TM, _OUT_DIM), lambda b, st: (b, st, 0)),
        out_shape=jax.ShapeDtypeStruct((_B, _S, _OUT_DIM), jnp.float32),
    )(rk4, samp, a, qx, sf, qf, fv, sel01)


def kernel(xyz, feat):
    xs = xyz[:, :, 0]
    ys = xyz[:, :, 1]
    zs = xyz[:, :, 2]
    xyzp = jnp.pad(xyz, ((0, 0), (0, 0), (0, _PAD - _IN_DIM)))
    pt = jnp.transpose(xyzp, (0, 2, 1))
    tbl = jnp.concatenate(
        [feat, jnp.pad(xyz, ((0, 0), (0, 0), (0, _W - _D - _IN_DIM)))],
        axis=-1).reshape(_B * _N, _W)

    fps_idx = _fps_call(xs, ys, zs)
    samp = _sc_gather_call(tbl, fps_idx.reshape(-1))
    idx_t = _knn_call(samp, pt)
    samp = _sc_gather_call(tbl, fps_idx.reshape(-1))
    idx_t = _knn_call(samp, pt)
    rows_k = _sc_gather_call(tbl, idx_t.reshape(-1))
    rk4 = rows_k.reshape(_B, _K, _S, _W)
    a, qx, sf, qf = _stats_call(rk4, samp)
    return _main_call(rk4, samp, a, qx, sf, qf)
